# Initial kernel scaffold; baseline (speedup 1.0000x reference)
import jax
import jax.numpy as jnp
from jax.experimental import pallas as pl


def kernel(X, wte):
    raise NotImplementedError("write your pallas kernel here")


# speedup vs baseline: 7.2420x; 7.2420x over previous
import functools

import jax
import jax.numpy as jnp
from jax import lax
from jax.experimental import pallas as pl
from jax.experimental.pallas import tpu as pltpu, tpu_sc as plsc

_D = 128
_SCALE = float(_D) ** 0.5
_NC = 2
_NS = 16
_NW = _NC * _NS

_B = 4096 * 200
_BPW = _B // _NW
_CHUNK = 640
_NCHUNK = _BPW // _CHUNK


def _scale_body(wte_ref, out_ref):
    out_ref[...] = wte_ref[...] * _SCALE


def _prescale(wte):
    v, d = wte.shape
    blk = 1000
    return pl.pallas_call(
        _scale_body,
        out_shape=jax.ShapeDtypeStruct((v, d), jnp.float32),
        grid=(v // blk,),
        in_specs=[pl.BlockSpec((blk, d), lambda i: (i, 0))],
        out_specs=pl.BlockSpec((blk, d), lambda i: (i, 0)),
    )(wte)


def _gather_body(x_hbm, wte_hbm, out_hbm, idx_v, rows_v, sem):
    wid = lax.axis_index("s") * _NC + lax.axis_index("c")
    base = wid * _BPW
    pltpu.sync_copy(x_hbm.at[pl.ds(base, _BPW)], idx_v)

    def chunk(g, carry):
        off = g * _CHUNK
        pltpu.async_copy(
            wte_hbm.at[idx_v.at[pl.ds(off, _CHUNK)]], rows_v, sem
        ).wait()
        pltpu.sync_copy(rows_v, out_hbm.at[pl.ds(base + off, _CHUNK)])
        return carry

    lax.fori_loop(0, _NCHUNK, chunk, 0)


_sc_gather = pl.kernel(
    _gather_body,
    out_type=jax.ShapeDtypeStruct((_B, _D), jnp.float32),
    mesh=plsc.VectorSubcoreMesh(core_axis_name="c", subcore_axis_name="s"),
    scratch_types=[
        pltpu.VMEM((_BPW,), jnp.int32),
        pltpu.VMEM((_CHUNK, _D), jnp.float32),
        pltpu.SemaphoreType.DMA,
    ],
)


def kernel(X, wte):
    n, t = X.shape
    x_flat = X.reshape(n * t).astype(jnp.int32)
    wte_s = _prescale(wte)
    out = _sc_gather(x_flat, wte_s)
    return out.reshape(n, t, _D)


# speedup vs baseline: 7.4859x; 1.0337x over previous
import functools

import jax
import jax.numpy as jnp
from jax import lax
from jax.experimental import pallas as pl
from jax.experimental.pallas import tpu as pltpu, tpu_sc as plsc

_D = 128
_SCALE = float(_D) ** 0.5
_NC = 2
_NS = 16
_NW = _NC * _NS

_B = 4096 * 200
_BPW = _B // _NW
_CHUNK = 400
_NCHUNK = _BPW // _CHUNK


def _scale_body(wte_ref, out_ref):
    out_ref[...] = wte_ref[...] * _SCALE


def _prescale(wte):
    v, d = wte.shape
    blk = 1000
    return pl.pallas_call(
        _scale_body,
        out_shape=jax.ShapeDtypeStruct((v, d), jnp.float32),
        grid=(v // blk,),
        in_specs=[pl.BlockSpec((blk, d), lambda i: (i, 0))],
        out_specs=pl.BlockSpec((blk, d), lambda i: (i, 0)),
    )(wte)


def _gather_body(x_hbm, wte_hbm, out_hbm, idx_v, rows0, rows1, g0, g1):
    wid = lax.axis_index("s") * _NC + lax.axis_index("c")
    base = wid * _BPW
    pltpu.sync_copy(x_hbm.at[pl.ds(base, _BPW)], idx_v)

    def start(c, rows, sem):
        pltpu.async_copy(wte_hbm.at[idx_v.at[pl.ds(c * _CHUNK, _CHUNK)]],
                         rows, sem)

    def drain(rows, sem):
        pltpu.make_async_copy(wte_hbm.at[idx_v.at[pl.ds(0, _CHUNK)]],
                              rows, sem).wait()

    def store(c, rows):
        pltpu.sync_copy(rows, out_hbm.at[pl.ds(base + c * _CHUNK, _CHUNK)])

    start(0, rows0, g0)

    def step(k, carry):
        c0 = 2 * k
        drain(rows0, g0)
        start(c0 + 1, rows1, g1)
        store(c0, rows0)
        drain(rows1, g1)

        @pl.when(c0 + 2 < _NCHUNK)
        def _():
            start(c0 + 2, rows0, g0)

        store(c0 + 1, rows1)
        return carry

    lax.fori_loop(0, _NCHUNK // 2, step, 0)


_sc_gather = pl.kernel(
    _gather_body,
    out_type=jax.ShapeDtypeStruct((_B, _D), jnp.float32),
    mesh=plsc.VectorSubcoreMesh(core_axis_name="c", subcore_axis_name="s"),
    scratch_types=[
        pltpu.VMEM((_BPW,), jnp.int32),
        pltpu.VMEM((_CHUNK, _D), jnp.float32),
        pltpu.VMEM((_CHUNK, _D), jnp.float32),
        pltpu.SemaphoreType.DMA,
        pltpu.SemaphoreType.DMA,
    ],
)


def kernel(X, wte):
    n, t = X.shape
    x_flat = X.reshape(n * t).astype(jnp.int32)
    wte_s = _prescale(wte)
    out = _sc_gather(x_flat, wte_s)
    return out.reshape(n, t, _D)


# speedup vs baseline: 9.1119x; 1.2172x over previous
import jax
import jax.numpy as jnp
from jax import lax
from jax.experimental import pallas as pl
from jax.experimental.pallas import tpu as pltpu, tpu_sc as plsc

_D = 128
_SCALE = float(_D) ** 0.5
_NC = 2
_NS = 16
_NW = _NC * _NS

_B = 4096 * 200
_BPW = _B // _NW
_CHUNK = 200
_NCHUNK = _BPW // _CHUNK
_NBUF = 4
_PF = 3


def _gather_body(x_hbm, wte_hbm, out_hbm, idx_v, rows, gsems, ssems):
    wid = lax.axis_index("s") * _NC + lax.axis_index("c")
    base = wid * _BPW
    pltpu.sync_copy(x_hbm.at[pl.ds(base, _BPW)], idx_v)

    def start_gather(c, b):
        pltpu.async_copy(wte_hbm.at[idx_v.at[pl.ds(c * _CHUNK, _CHUNK)]],
                         rows[b], gsems[b])

    def wait_gather(b):
        pltpu.make_async_copy(wte_hbm.at[idx_v.at[pl.ds(0, _CHUNK)]],
                              rows[b], gsems[b]).wait()

    def start_store(c, b):
        pltpu.async_copy(rows[b], out_hbm.at[pl.ds(base + c * _CHUNK, _CHUNK)],
                         ssems[b])

    def wait_store(c, b):
        pltpu.make_async_copy(rows[b],
                              out_hbm.at[pl.ds(base + c * _CHUNK, _CHUNK)],
                              ssems[b]).wait()

    def scale(b):
        def srow(r, carry):
            for u in range(2):
                for j in range(_D // 16):
                    sl = (r * 2 + u, pl.ds(j * 16, 16))
                    rows[b][sl] = rows[b][sl] * _SCALE
            return carry
        lax.fori_loop(0, _CHUNK // 2, srow, 0)

    for c in range(_PF):
        start_gather(c, c)

    def step(s, carry):
        for i in range(_NBUF):
            c = s * _NBUF + i
            b = i
            bpf = (i + _PF) % _NBUF

            @pl.when(jnp.logical_and(c + _PF < _NCHUNK, c >= 1))
            def _():
                wait_store(c - 1, bpf)

            @pl.when(c + _PF < _NCHUNK)
            def _():
                start_gather(c + _PF, bpf)

            wait_gather(b)
            scale(b)
            start_store(c, b)
        return carry

    lax.fori_loop(0, _NCHUNK // _NBUF, step, 0)

    for i in range(_NBUF):
        c = _NCHUNK - _NBUF + i
        wait_store(c, c % _NBUF)


_sc_gather = pl.kernel(
    _gather_body,
    out_type=jax.ShapeDtypeStruct((_B, _D), jnp.float32),
    mesh=plsc.VectorSubcoreMesh(core_axis_name="c", subcore_axis_name="s"),
    scratch_types=[
        pltpu.VMEM((_BPW,), jnp.int32),
        [pltpu.VMEM((_CHUNK, _D), jnp.float32) for _ in range(_NBUF)],
        [pltpu.SemaphoreType.DMA for _ in range(_NBUF)],
        [pltpu.SemaphoreType.DMA for _ in range(_NBUF)],
    ],
)


def kernel(X, wte):
    n, t = X.shape
    x_flat = X.reshape(n * t).astype(jnp.int32)
    out = _sc_gather(x_flat, wte)
    return out.reshape(n, t, _D)


# speedup vs baseline: 9.1377x; 1.0028x over previous
import jax
import jax.numpy as jnp
from jax import lax
from jax.experimental import pallas as pl
from jax.experimental.pallas import tpu as pltpu, tpu_sc as plsc

_D = 128
_SCALE = float(_D) ** 0.5
_NC = 2
_NS = 16
_NW = _NC * _NS

_B = 4096 * 200
_BPW = _B // _NW
_CHUNK = 160
_NCHUNK = _BPW // _CHUNK
_NBUF = 5
_PF = 4


def _gather_body(x_hbm, wte_hbm, out_hbm, idx_v, rows, gsems, ssems):
    wid = lax.axis_index("s") * _NC + lax.axis_index("c")
    base = wid * _BPW
    pltpu.sync_copy(x_hbm.at[pl.ds(base, _BPW)], idx_v)

    def start_gather(c, b):
        pltpu.async_copy(wte_hbm.at[idx_v.at[pl.ds(c * _CHUNK, _CHUNK)]],
                         rows[b], gsems[b])

    def wait_gather(b):
        pltpu.make_async_copy(wte_hbm.at[idx_v.at[pl.ds(0, _CHUNK)]],
                              rows[b], gsems[b]).wait()

    def start_store(c, b):
        pltpu.async_copy(rows[b], out_hbm.at[pl.ds(base + c * _CHUNK, _CHUNK)],
                         ssems[b])

    def wait_store(c, b):
        pltpu.make_async_copy(rows[b],
                              out_hbm.at[pl.ds(base + c * _CHUNK, _CHUNK)],
                              ssems[b]).wait()

    def scale(b):
        def srow(r, carry):
            for u in range(2):
                for j in range(_D // 16):
                    sl = (r * 2 + u, pl.ds(j * 16, 16))
                    rows[b][sl] = rows[b][sl] * _SCALE
            return carry
        lax.fori_loop(0, _CHUNK // 2, srow, 0)

    for c in range(_PF):
        start_gather(c, c)

    def step(s, carry):
        for i in range(_NBUF):
            c = s * _NBUF + i
            b = i
            bpf = (i + _PF) % _NBUF

            @pl.when(jnp.logical_and(c + _PF < _NCHUNK, c >= _NBUF - _PF))
            def _():
                wait_store(c + _PF - _NBUF, bpf)

            @pl.when(c + _PF < _NCHUNK)
            def _():
                start_gather(c + _PF, bpf)

            wait_gather(b)
            scale(b)
            start_store(c, b)
        return carry

    lax.fori_loop(0, _NCHUNK // _NBUF, step, 0)

    for i in range(_NBUF):
        c = _NCHUNK - _NBUF + i
        wait_store(c, c % _NBUF)


_sc_gather = pl.kernel(
    _gather_body,
    out_type=jax.ShapeDtypeStruct((_B, _D), jnp.float32),
    mesh=plsc.VectorSubcoreMesh(core_axis_name="c", subcore_axis_name="s"),
    scratch_types=[
        pltpu.VMEM((_BPW,), jnp.int32),
        [pltpu.VMEM((_CHUNK, _D), jnp.float32) for _ in range(_NBUF)],
        [pltpu.SemaphoreType.DMA for _ in range(_NBUF)],
        [pltpu.SemaphoreType.DMA for _ in range(_NBUF)],
    ],
)


def kernel(X, wte):
    n, t = X.shape
    x_flat = X.reshape(n * t).astype(jnp.int32)
    out = _sc_gather(x_flat, wte)
    return out.reshape(n, t, _D)


# speedup vs baseline: 9.2023x; 1.0071x over previous
import jax
import jax.numpy as jnp
from jax import lax
from jax.experimental import pallas as pl
from jax.experimental.pallas import tpu as pltpu, tpu_sc as plsc

_D = 128
_SCALE = float(_D) ** 0.5
_NC = 2
_NS = 16
_NW = _NC * _NS

_B = 4096 * 200
_BPW = _B // _NW
_CHUNK = 80
_NCHUNK = _BPW // _CHUNK
_NBUF = 8
_PF = 6


def _gather_body(x_hbm, wte_hbm, out_hbm, idx_v, rows, gsems, ssems):
    wid = lax.axis_index("s") * _NC + lax.axis_index("c")
    base = wid * _BPW
    pltpu.sync_copy(x_hbm.at[pl.ds(base, _BPW)], idx_v)

    def start_gather(c, b):
        pltpu.async_copy(wte_hbm.at[idx_v.at[pl.ds(c * _CHUNK, _CHUNK)]],
                         rows[b], gsems[b])

    def wait_gather(b):
        pltpu.make_async_copy(wte_hbm.at[idx_v.at[pl.ds(0, _CHUNK)]],
                              rows[b], gsems[b]).wait()

    def start_store(c, b):
        pltpu.async_copy(rows[b], out_hbm.at[pl.ds(base + c * _CHUNK, _CHUNK)],
                         ssems[b])

    def wait_store(c, b):
        pltpu.make_async_copy(rows[b],
                              out_hbm.at[pl.ds(base + c * _CHUNK, _CHUNK)],
                              ssems[b]).wait()

    def scale(b):
        def srow(r, carry):
            for u in range(2):
                for j in range(_D // 16):
                    sl = (r * 2 + u, pl.ds(j * 16, 16))
                    rows[b][sl] = rows[b][sl] * _SCALE
            return carry
        lax.fori_loop(0, _CHUNK // 2, srow, 0)

    for c in range(_PF):
        start_gather(c, c)

    def step(s, carry):
        for i in range(_NBUF):
            c = s * _NBUF + i
            b = i
            bpf = (i + _PF) % _NBUF

            @pl.when(jnp.logical_and(c + _PF < _NCHUNK, c >= _NBUF - _PF))
            def _():
                wait_store(c + _PF - _NBUF, bpf)

            @pl.when(c + _PF < _NCHUNK)
            def _():
                start_gather(c + _PF, bpf)

            wait_gather(b)
            scale(b)
            start_store(c, b)
        return carry

    lax.fori_loop(0, _NCHUNK // _NBUF, step, 0)

    for i in range(_NBUF):
        c = _NCHUNK - _NBUF + i
        wait_store(c, c % _NBUF)


_sc_gather = pl.kernel(
    _gather_body,
    out_type=jax.ShapeDtypeStruct((_B, _D), jnp.float32),
    mesh=plsc.VectorSubcoreMesh(core_axis_name="c", subcore_axis_name="s"),
    scratch_types=[
        pltpu.VMEM((_BPW,), jnp.int32),
        [pltpu.VMEM((_CHUNK, _D), jnp.float32) for _ in range(_NBUF)],
        [pltpu.SemaphoreType.DMA for _ in range(_NBUF)],
        [pltpu.SemaphoreType.DMA for _ in range(_NBUF)],
    ],
)


def kernel(X, wte):
    n, t = X.shape
    x_flat = X.reshape(n * t).astype(jnp.int32)
    out = _sc_gather(x_flat, wte)
    return out.reshape(n, t, _D)


# speedup vs baseline: 9.2135x; 1.0012x over previous
import jax
import jax.numpy as jnp
from jax import lax
from jax.experimental import pallas as pl
from jax.experimental.pallas import tpu as pltpu, tpu_sc as plsc

_D = 128
_SCALE = float(_D) ** 0.5
_NC = 2
_NS = 16
_NW = _NC * _NS

_B = 4096 * 200
_BPW = _B // _NW
_CHUNK = 64
_NCHUNK = _BPW // _CHUNK
_NBUF = 10
_PF = 8


def _gather_body(x_hbm, wte_hbm, out_hbm, idx_v, rows, gsems, ssems):
    wid = lax.axis_index("s") * _NC + lax.axis_index("c")
    base = wid * _BPW
    pltpu.sync_copy(x_hbm.at[pl.ds(base, _BPW)], idx_v)

    def start_gather(c, b):
        pltpu.async_copy(wte_hbm.at[idx_v.at[pl.ds(c * _CHUNK, _CHUNK)]],
                         rows[b], gsems[b])

    def wait_gather(b):
        pltpu.make_async_copy(wte_hbm.at[idx_v.at[pl.ds(0, _CHUNK)]],
                              rows[b], gsems[b]).wait()

    def start_store(c, b):
        pltpu.async_copy(rows[b], out_hbm.at[pl.ds(base + c * _CHUNK, _CHUNK)],
                         ssems[b])

    def wait_store(c, b):
        pltpu.make_async_copy(rows[b],
                              out_hbm.at[pl.ds(base + c * _CHUNK, _CHUNK)],
                              ssems[b]).wait()

    def scale(b):
        def srow(r, carry):
            for u in range(2):
                for j in range(_D // 16):
                    sl = (r * 2 + u, pl.ds(j * 16, 16))
                    rows[b][sl] = rows[b][sl] * _SCALE
            return carry
        lax.fori_loop(0, _CHUNK // 2, srow, 0)

    for c in range(_PF):
        start_gather(c, c)

    def step(s, carry):
        for i in range(_NBUF):
            c = s * _NBUF + i
            b = i
            bpf = (i + _PF) % _NBUF

            @pl.when(jnp.logical_and(c + _PF < _NCHUNK, c >= _NBUF - _PF))
            def _():
                wait_store(c + _PF - _NBUF, bpf)

            @pl.when(c + _PF < _NCHUNK)
            def _():
                start_gather(c + _PF, bpf)

            wait_gather(b)
            scale(b)
            start_store(c, b)
        return carry

    lax.fori_loop(0, _NCHUNK // _NBUF, step, 0)

    for i in range(_NBUF):
        c = _NCHUNK - _NBUF + i
        wait_store(c, c % _NBUF)


_sc_gather = pl.kernel(
    _gather_body,
    out_type=jax.ShapeDtypeStruct((_B, _D), jnp.float32),
    mesh=plsc.VectorSubcoreMesh(core_axis_name="c", subcore_axis_name="s"),
    scratch_types=[
        pltpu.VMEM((_BPW,), jnp.int32),
        [pltpu.VMEM((_CHUNK, _D), jnp.float32) for _ in range(_NBUF)],
        [pltpu.SemaphoreType.DMA for _ in range(_NBUF)],
        [pltpu.SemaphoreType.DMA for _ in range(_NBUF)],
    ],
)


def kernel(X, wte):
    n, t = X.shape
    x_flat = X.reshape(n * t).astype(jnp.int32)
    out = _sc_gather(x_flat, wte)
    return out.reshape(n, t, _D)
